# Initial kernel scaffold; baseline (speedup 1.0000x reference)
import jax
import jax.numpy as jnp
from jax.experimental import pallas as pl


def kernel(x, edge_index, W1, b1, w_att, W2, b2):
    raise NotImplementedError("write your pallas kernel here")


# speedup vs baseline: 20.0439x; 20.0439x over previous
import functools

import jax
import jax.numpy as jnp
from jax import lax
from jax.experimental import pallas as pl
from jax.experimental.pallas import tpu as pltpu
from jax.experimental.pallas import tpu_sc as plsc

N = 50000
E = 1600000
F = 16
DEG_LANE = 11
NC = 2
NS = 16
NW = NC * NS
ROW = 128
RPW = (-(-E // (NW * ROW)) + 7) // 8 * 8
CH = 56
E_PAD = NW * ROW * RPW
N_ACC = ((N + NS - 1) // NS + 7) // 8 * 8 * NS
RPS = N_ACC // NS


def _sc_segment_sum(xp, zeros_acc, src2d, dst2d):
    mesh = plsc.VectorSubcoreMesh(
        core_axis_name="c", subcore_axis_name="s", num_cores=NC)

    @functools.partial(
        pl.kernel,
        out_type=jax.ShapeDtypeStruct((NC, N_ACC, F), jnp.float32),
        mesh=mesh,
        scratch_types=[
            pltpu.VMEM_SHARED((N_ACC, F), jnp.float32),
            pltpu.VMEM((CH, ROW), jnp.int32),
            pltpu.VMEM((CH, ROW), jnp.int32),
            pltpu.VMEM((ROW, F), jnp.float32),
            pltpu.SemaphoreType.DMA,
        ],
        compiler_params=pltpu.CompilerParams(use_tc_tiling_on_sc=False),
    )
    def sc_kernel(xp_hbm, zer_hbm, src_hbm, dst_hbm, out_hbm,
                  acc, srcv, dstv, rows, sem):
        c = lax.axis_index("c")
        s = lax.axis_index("s")
        w = s * NC + c
        pltpu.sync_copy(zer_hbm.at[pl.ds(s * RPS, RPS)],
                        acc.at[pl.ds(s * RPS, RPS)])
        plsc.subcore_barrier()

        def chunk(k, carry):
            base = w * RPW + k * CH
            pltpu.sync_copy(src_hbm.at[pl.ds(base, CH)], srcv)
            pltpu.sync_copy(dst_hbm.at[pl.ds(base, CH)], dstv)

            def step(j, carry2):
                pltpu.async_copy(xp_hbm.at[srcv.at[j]], rows, sem).wait()
                pltpu.sync_copy(rows, acc.at[dstv.at[j]], add=True)
                return carry2

            return lax.fori_loop(0, CH, step, carry)

        lax.fori_loop(0, RPW // CH, chunk, 0)
        plsc.subcore_barrier()
        pltpu.sync_copy(acc.at[pl.ds(s * RPS, RPS)],
                        out_hbm.at[c, pl.ds(s * RPS, RPS)])

    return sc_kernel(xp, zeros_acc, src2d, dst2d)


FOLD = 8
NR = N_ACC * F // 128


def _tc_head(part_f, xp_f, w_big, b1_tile, watt_blk, esel, bbrd, w2_blk, b2):
    def tc_kernel(part_ref, xp_ref, wb_ref, b1_ref, wa_ref, es_ref, bb_ref,
                  w2_ref, b2_ref, out_ref):
        acc = part_ref[0] + part_ref[1] + xp_ref[...]
        den8 = jnp.dot(acc, es_ref[...],
                       preferred_element_type=jnp.float32)
        den = jnp.dot(den8, bb_ref[...],
                      preferred_element_type=jnp.float32)
        den = jnp.maximum(den, 0.5)
        num = jnp.dot(acc, wb_ref[...], preferred_element_type=jnp.float32)
        h = jnp.maximum(num / den + b1_ref[...], 0.0)
        t8 = jnp.tanh(jnp.dot(h, wa_ref[...],
                              preferred_element_type=jnp.float32))
        rid = lax.broadcasted_iota(jnp.int32, (NR, FOLD), 0)
        cid = lax.broadcasted_iota(jnp.int32, (NR, FOLD), 1)
        valid = rid * FOLD + cid < N
        t8 = jnp.where(valid, t8, -30.0)
        m = jnp.max(t8)
        e = jnp.exp(t8 - m)
        scores = e / jnp.sum(e)
        o8 = jnp.dot(h, w2_ref[...], preferred_element_type=jnp.float32)
        out_ref[...] = o8 * scores + b2_ref[...]

    return pl.pallas_call(
        tc_kernel,
        out_shape=jax.ShapeDtypeStruct((NR, FOLD), jnp.float32),
    )(part_f, xp_f, w_big, b1_tile, watt_blk, esel, bbrd, w2_blk, b2)


def kernel(x, edge_index, W1, b1, w_att, W2, b2):
    xp = jnp.concatenate(
        [x, jnp.ones((N, 1), jnp.float32), jnp.zeros((N, F - 12), jnp.float32)],
        axis=1)
    xp = jnp.pad(xp, ((0, N_ACC - N), (0, 0)))
    src = jnp.pad(edge_index[0], (0, E_PAD - E))
    dst = jnp.pad(edge_index[1], (0, E_PAD - E),
                  constant_values=N)
    src2d = src.reshape(NW * RPW, ROW)
    dst2d = dst.reshape(NW * RPW, ROW)
    zeros_acc = jnp.zeros((N_ACC, F), jnp.float32)

    part = _sc_segment_sum(xp, zeros_acc, src2d, dst2d)

    w1p = jnp.zeros((F, F), jnp.float32).at[:11, :].set(W1)
    eye8 = jnp.eye(FOLD, dtype=jnp.float32)
    w_big = jnp.kron(eye8, w1p)
    watt_blk = jnp.kron(eye8, w_att.reshape(F, 1))
    esel = jnp.kron(eye8, jnp.zeros((F, 1), jnp.float32)
                    .at[DEG_LANE, 0].set(1.0))
    bbrd = jnp.kron(eye8, jnp.ones((1, F), jnp.float32))
    w2_blk = jnp.kron(eye8, W2)
    b1_tile = jnp.tile(b1, FOLD).reshape(1, 128)

    out = _tc_head(part.reshape(NC, NR, 128), xp.reshape(NR, 128),
                   w_big, b1_tile, watt_blk, esel, bbrd, w2_blk,
                   b2.reshape(1, 1))
    return out.reshape(N_ACC)[:N]


# speedup vs baseline: 35.0765x; 1.7500x over previous
import functools

import jax
import jax.numpy as jnp
from jax import lax
from jax.experimental import pallas as pl
from jax.experimental.pallas import tpu as pltpu
from jax.experimental.pallas import tpu_sc as plsc

N = 50000
E = 1600000
F = 16
DEG_LANE = 11
NC = 2
NS = 16
NW = NC * NS
ROW = 128
RPW = (-(-E // (NW * ROW)) + 7) // 8 * 8
CH = 56
K = 8
E_PAD = NW * ROW * RPW
N_ACC = ((N + NS - 1) // NS + 7) // 8 * 8 * NS
RPS = N_ACC // NS


def _sc_segment_sum(xp, zeros_acc, src2d, dst2d):
    mesh = plsc.VectorSubcoreMesh(
        core_axis_name="c", subcore_axis_name="s", num_cores=NC)

    @functools.partial(
        pl.kernel,
        out_type=jax.ShapeDtypeStruct((NC, N_ACC, F), jnp.float32),
        mesh=mesh,
        scratch_types=[
            pltpu.VMEM_SHARED((N_ACC, F), jnp.float32),
            pltpu.VMEM((CH, ROW), jnp.int32),
            pltpu.VMEM((CH, ROW), jnp.int32),
            pltpu.VMEM((K, ROW, F), jnp.float32),
            pltpu.SemaphoreType.DMA,
            pltpu.SemaphoreType.DMA,
        ],
        compiler_params=pltpu.CompilerParams(use_tc_tiling_on_sc=False),
    )
    def sc_kernel(xp_hbm, zer_hbm, src_hbm, dst_hbm, out_hbm,
                  acc, srcv, dstv, rows, gsem, ssem):
        c = lax.axis_index("c")
        s = lax.axis_index("s")
        w = s * NC + c
        pltpu.sync_copy(zer_hbm.at[pl.ds(s * RPS, RPS)],
                        acc.at[pl.ds(s * RPS, RPS)])
        plsc.subcore_barrier()

        def chunk(k, carry):
            base = w * RPW + k * CH
            pltpu.sync_copy(src_hbm.at[pl.ds(base, CH)], srcv)
            pltpu.sync_copy(dst_hbm.at[pl.ds(base, CH)], dstv)

            def batch(jb, carry2):
                j0 = jb * K
                gd = [pltpu.async_copy(xp_hbm.at[srcv.at[j0 + b]],
                                       rows.at[b], gsem)
                      for b in range(K)]
                sd = []
                for b in range(K):
                    gd[b].wait()
                    sd.append(pltpu.async_copy(rows.at[b],
                                               acc.at[dstv.at[j0 + b]],
                                               ssem, add=True))
                for d in sd:
                    d.wait()
                return carry2

            return lax.fori_loop(0, CH // K, batch, carry)

        lax.fori_loop(0, RPW // CH, chunk, 0)
        plsc.subcore_barrier()
        pltpu.sync_copy(acc.at[pl.ds(s * RPS, RPS)],
                        out_hbm.at[c, pl.ds(s * RPS, RPS)])

    return sc_kernel(xp, zeros_acc, src2d, dst2d)


FOLD = 8
NR = N_ACC * F // 128


def _tc_head(part_f, xp_f, w_big, b1_tile, watt_blk, esel, bbrd, w2_blk, b2):
    def tc_kernel(part_ref, xp_ref, wb_ref, b1_ref, wa_ref, es_ref, bb_ref,
                  w2_ref, b2_ref, out_ref):
        acc = part_ref[0] + part_ref[1] + xp_ref[...]
        den8 = jnp.dot(acc, es_ref[...],
                       preferred_element_type=jnp.float32)
        den = jnp.dot(den8, bb_ref[...],
                      preferred_element_type=jnp.float32)
        den = jnp.maximum(den, 0.5)
        num = jnp.dot(acc, wb_ref[...], preferred_element_type=jnp.float32)
        h = jnp.maximum(num / den + b1_ref[...], 0.0)
        t8 = jnp.tanh(jnp.dot(h, wa_ref[...],
                              preferred_element_type=jnp.float32))
        rid = lax.broadcasted_iota(jnp.int32, (NR, FOLD), 0)
        cid = lax.broadcasted_iota(jnp.int32, (NR, FOLD), 1)
        valid = rid * FOLD + cid < N
        t8 = jnp.where(valid, t8, -30.0)
        m = jnp.max(t8)
        e = jnp.exp(t8 - m)
        scores = e / jnp.sum(e)
        o8 = jnp.dot(h, w2_ref[...], preferred_element_type=jnp.float32)
        out_ref[...] = o8 * scores + b2_ref[...]

    return pl.pallas_call(
        tc_kernel,
        out_shape=jax.ShapeDtypeStruct((NR, FOLD), jnp.float32),
    )(part_f, xp_f, w_big, b1_tile, watt_blk, esel, bbrd, w2_blk, b2)


def kernel(x, edge_index, W1, b1, w_att, W2, b2):
    xp = jnp.concatenate(
        [x, jnp.ones((N, 1), jnp.float32), jnp.zeros((N, F - 12), jnp.float32)],
        axis=1)
    xp = jnp.pad(xp, ((0, N_ACC - N), (0, 0)))
    src = jnp.pad(edge_index[0], (0, E_PAD - E))
    dst = jnp.pad(edge_index[1], (0, E_PAD - E),
                  constant_values=N)
    src2d = src.reshape(NW * RPW, ROW)
    dst2d = dst.reshape(NW * RPW, ROW)
    zeros_acc = jnp.zeros((N_ACC, F), jnp.float32)

    part = _sc_segment_sum(xp, zeros_acc, src2d, dst2d)

    w1p = jnp.zeros((F, F), jnp.float32).at[:11, :].set(W1)
    eye8 = jnp.eye(FOLD, dtype=jnp.float32)
    w_big = jnp.kron(eye8, w1p)
    watt_blk = jnp.kron(eye8, w_att.reshape(F, 1))
    esel = jnp.kron(eye8, jnp.zeros((F, 1), jnp.float32)
                    .at[DEG_LANE, 0].set(1.0))
    bbrd = jnp.kron(eye8, jnp.ones((1, F), jnp.float32))
    w2_blk = jnp.kron(eye8, W2)
    b1_tile = jnp.tile(b1, FOLD).reshape(1, 128)

    out = _tc_head(part.reshape(NC, NR, 128), xp.reshape(NR, 128),
                   w_big, b1_tile, watt_blk, esel, bbrd, w2_blk,
                   b2.reshape(1, 1))
    return out.reshape(N_ACC)[:N]


# speedup vs baseline: 40.9515x; 1.1675x over previous
import functools

import jax
import jax.numpy as jnp
from jax import lax
from jax.experimental import pallas as pl
from jax.experimental.pallas import tpu as pltpu
from jax.experimental.pallas import tpu_sc as plsc

N = 50000
E = 1600000
F = 16
DEG_LANE = 11
NC = 2
NS = 16
NW = NC * NS
ROW = 1250
RPW = E // (NW * ROW)
CH = 8
K = 2
N_ACC = ((N + NS - 1) // NS + 7) // 8 * 8 * NS
RPS = N_ACC // NS


def _sc_segment_sum(xp, zeros_acc, edge3):
    mesh = plsc.VectorSubcoreMesh(
        core_axis_name="c", subcore_axis_name="s", num_cores=NC)

    @functools.partial(
        pl.kernel,
        out_type=jax.ShapeDtypeStruct((NC, N_ACC, F), jnp.float32),
        mesh=mesh,
        scratch_types=[
            pltpu.VMEM_SHARED((N_ACC, F), jnp.float32),
            pltpu.VMEM((CH, ROW), jnp.int32),
            pltpu.VMEM((CH, ROW), jnp.int32),
            pltpu.VMEM((K, ROW, F), jnp.float32),
            pltpu.SemaphoreType.DMA,
            pltpu.SemaphoreType.DMA,
        ],
        compiler_params=pltpu.CompilerParams(use_tc_tiling_on_sc=False),
    )
    def sc_kernel(xp_hbm, zer_hbm, edge_hbm, out_hbm,
                  acc, srcv, dstv, rows, gsem, ssem):
        c = lax.axis_index("c")
        s = lax.axis_index("s")
        w = s * NC + c
        pltpu.sync_copy(zer_hbm.at[pl.ds(s * RPS, RPS)],
                        acc.at[pl.ds(s * RPS, RPS)])
        plsc.subcore_barrier()

        def chunk(k, carry):
            base = w * RPW + k * CH
            pltpu.sync_copy(edge_hbm.at[0, pl.ds(base, CH)], srcv)
            pltpu.sync_copy(edge_hbm.at[1, pl.ds(base, CH)], dstv)

            def batch(jb, carry2):
                j0 = jb * K
                gd = [pltpu.async_copy(xp_hbm.at[srcv.at[j0 + b]],
                                       rows.at[b], gsem)
                      for b in range(K)]
                sd = []
                for b in range(K):
                    gd[b].wait()
                    sd.append(pltpu.async_copy(rows.at[b],
                                               acc.at[dstv.at[j0 + b]],
                                               ssem, add=True))
                for d in sd:
                    d.wait()
                return carry2

            return lax.fori_loop(0, CH // K, batch, carry)

        lax.fori_loop(0, RPW // CH, chunk, 0)
        plsc.subcore_barrier()
        pltpu.sync_copy(acc.at[pl.ds(s * RPS, RPS)],
                        out_hbm.at[c, pl.ds(s * RPS, RPS)])

    return sc_kernel(xp, zeros_acc, edge3)


FOLD = 8
NR = N_ACC * F // 128


def _tc_head(part_f, xp_f, w_big, b1_tile, watt_blk, esel, bbrd, w2_blk, b2):
    def tc_kernel(part_ref, xp_ref, wb_ref, b1_ref, wa_ref, es_ref, bb_ref,
                  w2_ref, b2_ref, out_ref):
        acc = part_ref[0] + part_ref[1] + xp_ref[...]
        den8 = jnp.dot(acc, es_ref[...],
                       preferred_element_type=jnp.float32)
        den = jnp.dot(den8, bb_ref[...],
                      preferred_element_type=jnp.float32)
        den = jnp.maximum(den, 0.5)
        num = jnp.dot(acc, wb_ref[...], preferred_element_type=jnp.float32)
        h = jnp.maximum(num / den + b1_ref[...], 0.0)
        t8 = jnp.tanh(jnp.dot(h, wa_ref[...],
                              preferred_element_type=jnp.float32))
        rid = lax.broadcasted_iota(jnp.int32, (NR, FOLD), 0)
        cid = lax.broadcasted_iota(jnp.int32, (NR, FOLD), 1)
        valid = rid * FOLD + cid < N
        t8 = jnp.where(valid, t8, -30.0)
        m = jnp.max(t8)
        e = jnp.exp(t8 - m)
        scores = e / jnp.sum(e)
        o8 = jnp.dot(h, w2_ref[...], preferred_element_type=jnp.float32)
        out_ref[...] = o8 * scores + b2_ref[...]

    return pl.pallas_call(
        tc_kernel,
        out_shape=jax.ShapeDtypeStruct((NR, FOLD), jnp.float32),
    )(part_f, xp_f, w_big, b1_tile, watt_blk, esel, bbrd, w2_blk, b2)


def kernel(x, edge_index, W1, b1, w_att, W2, b2):
    xp = jnp.concatenate(
        [x, jnp.ones((N, 1), jnp.float32), jnp.zeros((N, F - 12), jnp.float32)],
        axis=1)
    xp = jnp.pad(xp, ((0, N_ACC - N), (0, 0)))
    edge3 = edge_index.reshape(2, NW * RPW, ROW)
    zeros_acc = jnp.zeros((N_ACC, F), jnp.float32)

    part = _sc_segment_sum(xp, zeros_acc, edge3)

    w1p = jnp.zeros((F, F), jnp.float32).at[:11, :].set(W1)
    eye8 = jnp.eye(FOLD, dtype=jnp.float32)
    w_big = jnp.kron(eye8, w1p)
    watt_blk = jnp.kron(eye8, w_att.reshape(F, 1))
    esel = jnp.kron(eye8, jnp.zeros((F, 1), jnp.float32)
                    .at[DEG_LANE, 0].set(1.0))
    bbrd = jnp.kron(eye8, jnp.ones((1, F), jnp.float32))
    w2_blk = jnp.kron(eye8, W2)
    b1_tile = jnp.tile(b1, FOLD).reshape(1, 128)

    out = _tc_head(part.reshape(NC, NR, 128), xp.reshape(NR, 128),
                   w_big, b1_tile, watt_blk, esel, bbrd, w2_blk,
                   b2.reshape(1, 1))
    return out.reshape(N_ACC)[:N]


# speedup vs baseline: 48.4426x; 1.1829x over previous
import functools

import jax
import jax.numpy as jnp
from jax import lax
from jax.experimental import pallas as pl
from jax.experimental.pallas import tpu as pltpu
from jax.experimental.pallas import tpu_sc as plsc

N = 50000
E = 1600000
F = 16
DEG_LANE = 11
NC = 2
NS = 16
NW = NC * NS
ROW = 1000
CH = 10
CHE = CH * ROW
EPW = E // NW
NCHUNK = EPW // CHE
K = 2
N_ACC = ((N + NS - 1) // NS + 7) // 8 * 8 * NS
RPS = N_ACC // NS


def _sc_segment_sum(xp, zeros_acc, edge3):
    mesh = plsc.VectorSubcoreMesh(
        core_axis_name="c", subcore_axis_name="s", num_cores=NC)

    @functools.partial(
        pl.kernel,
        out_type=jax.ShapeDtypeStruct((NC, N_ACC, F), jnp.float32),
        mesh=mesh,
        scratch_types=[
            pltpu.VMEM_SHARED((N_ACC, F), jnp.float32),
            pltpu.VMEM((CHE,), jnp.int32),
            pltpu.VMEM((CHE,), jnp.int32),
            pltpu.VMEM((K, ROW, F), jnp.float32),
            pltpu.SemaphoreType.DMA,
            pltpu.SemaphoreType.DMA,
        ],
        compiler_params=pltpu.CompilerParams(use_tc_tiling_on_sc=False),
    )
    def sc_kernel(xp_hbm, zer_hbm, edge_hbm, out_hbm,
                  acc, srcv, dstv, rows, gsem, ssem):
        c = lax.axis_index("c")
        s = lax.axis_index("s")
        w = s * NC + c
        pltpu.sync_copy(zer_hbm.at[pl.ds(s * RPS, RPS)],
                        acc.at[pl.ds(s * RPS, RPS)])
        plsc.subcore_barrier()

        def chunk(k, carry):
            base = w * EPW + k * CHE
            pltpu.sync_copy(edge_hbm.at[0, pl.ds(base, CHE)], srcv)
            pltpu.sync_copy(edge_hbm.at[1, pl.ds(base, CHE)], dstv)

            def batch(jb, carry2):
                j0 = jb * K
                gd = [pltpu.async_copy(
                        xp_hbm.at[srcv.at[pl.ds((j0 + b) * ROW, ROW)]],
                        rows.at[b], gsem)
                      for b in range(K)]
                sd = []
                for b in range(K):
                    gd[b].wait()
                    sd.append(pltpu.async_copy(
                        rows.at[b],
                        acc.at[dstv.at[pl.ds((j0 + b) * ROW, ROW)]],
                        ssem, add=True))
                for d in sd:
                    d.wait()
                return carry2

            return lax.fori_loop(0, CH // K, batch, carry)

        lax.fori_loop(0, NCHUNK, chunk, 0)
        plsc.subcore_barrier()
        pltpu.sync_copy(acc.at[pl.ds(s * RPS, RPS)],
                        out_hbm.at[c, pl.ds(s * RPS, RPS)])

    return sc_kernel(xp, zeros_acc, edge3)


FOLD = 8
NR = N_ACC * F // 128


def _tc_head(part_f, xp_f, w_big, b1_tile, watt_blk, esel, bbrd, w2_blk, b2):
    def tc_kernel(part_ref, xp_ref, wb_ref, b1_ref, wa_ref, es_ref, bb_ref,
                  w2_ref, b2_ref, out_ref):
        acc = part_ref[0] + part_ref[1] + xp_ref[...]
        den8 = jnp.dot(acc, es_ref[...],
                       preferred_element_type=jnp.float32)
        den = jnp.dot(den8, bb_ref[...],
                      preferred_element_type=jnp.float32)
        den = jnp.maximum(den, 0.5)
        num = jnp.dot(acc, wb_ref[...], preferred_element_type=jnp.float32)
        h = jnp.maximum(num / den + b1_ref[...], 0.0)
        t8 = jnp.tanh(jnp.dot(h, wa_ref[...],
                              preferred_element_type=jnp.float32))
        rid = lax.broadcasted_iota(jnp.int32, (NR, FOLD), 0)
        cid = lax.broadcasted_iota(jnp.int32, (NR, FOLD), 1)
        valid = rid * FOLD + cid < N
        t8 = jnp.where(valid, t8, -30.0)
        m = jnp.max(t8)
        e = jnp.exp(t8 - m)
        scores = e / jnp.sum(e)
        o8 = jnp.dot(h, w2_ref[...], preferred_element_type=jnp.float32)
        out_ref[...] = o8 * scores + b2_ref[...]

    return pl.pallas_call(
        tc_kernel,
        out_shape=jax.ShapeDtypeStruct((NR, FOLD), jnp.float32),
    )(part_f, xp_f, w_big, b1_tile, watt_blk, esel, bbrd, w2_blk, b2)


def kernel(x, edge_index, W1, b1, w_att, W2, b2):
    xp = jnp.concatenate(
        [x, jnp.ones((N, 1), jnp.float32), jnp.zeros((N, F - 12), jnp.float32)],
        axis=1)
    xp = jnp.pad(xp, ((0, N_ACC - N), (0, 0)))
    zeros_acc = jnp.zeros((N_ACC, F), jnp.float32)

    part = _sc_segment_sum(xp, zeros_acc, edge_index)

    w1p = jnp.zeros((F, F), jnp.float32).at[:11, :].set(W1)
    eye8 = jnp.eye(FOLD, dtype=jnp.float32)
    w_big = jnp.kron(eye8, w1p)
    watt_blk = jnp.kron(eye8, w_att.reshape(F, 1))
    esel = jnp.kron(eye8, jnp.zeros((F, 1), jnp.float32)
                    .at[DEG_LANE, 0].set(1.0))
    bbrd = jnp.kron(eye8, jnp.ones((1, F), jnp.float32))
    w2_blk = jnp.kron(eye8, W2)
    b1_tile = jnp.tile(b1, FOLD).reshape(1, 128)

    out = _tc_head(part.reshape(NC, NR, 128), xp.reshape(NR, 128),
                   w_big, b1_tile, watt_blk, esel, bbrd, w2_blk,
                   b2.reshape(1, 1))
    return out.reshape(N_ACC)[:N]


# speedup vs baseline: 52.0694x; 1.0749x over previous
import functools

import jax
import jax.numpy as jnp
from jax import lax
from jax.experimental import pallas as pl
from jax.experimental.pallas import tpu as pltpu
from jax.experimental.pallas import tpu_sc as plsc

N = 50000
E = 1600000
F = 16
DEG_LANE = 11
NC = 2
NS = 16
NW = NC * NS
ROW = 400
CH = 25
CHE = CH * ROW
EPW = E // NW
NCHUNK = EPW // CHE
K = 5
N_ACC = ((N + NS - 1) // NS + 7) // 8 * 8 * NS
RPS = N_ACC // NS


def _sc_segment_sum(xp, zeros_acc, edge3):
    mesh = plsc.VectorSubcoreMesh(
        core_axis_name="c", subcore_axis_name="s", num_cores=NC)

    @functools.partial(
        pl.kernel,
        out_type=jax.ShapeDtypeStruct((NC, N_ACC, F), jnp.float32),
        mesh=mesh,
        scratch_types=[
            pltpu.VMEM_SHARED((N_ACC, F), jnp.float32),
            pltpu.VMEM((2, CHE), jnp.int32),
            pltpu.VMEM((2, CHE), jnp.int32),
            pltpu.VMEM((K, ROW, F), jnp.float32),
            pltpu.SemaphoreType.DMA,
            pltpu.SemaphoreType.DMA,
            pltpu.SemaphoreType.DMA,
            pltpu.SemaphoreType.DMA,
        ],
        compiler_params=pltpu.CompilerParams(use_tc_tiling_on_sc=False),
    )
    def sc_kernel(xp_hbm, zer_hbm, edge_hbm, out_hbm,
                  acc, srcv, dstv, rows, gsem, ssem, esem, zsem):
        c = lax.axis_index("c")
        s = lax.axis_index("s")
        w = s * NC + c
        zd = pltpu.async_copy(zer_hbm.at[pl.ds(s * RPS, RPS)],
                              acc.at[pl.ds(s * RPS, RPS)], zsem)

        def stage(k, buf):
            base = w * EPW + k * CHE
            return [pltpu.async_copy(edge_hbm.at[0, pl.ds(base, CHE)],
                                     srcv.at[buf], esem),
                    pltpu.async_copy(edge_hbm.at[1, pl.ds(base, CHE)],
                                     dstv.at[buf], esem)]

        pending = stage(0, 0)
        zd.wait()
        plsc.subcore_barrier()

        for k in range(NCHUNK):
            buf = k % 2
            for d in pending:
                d.wait()
            if k + 1 < NCHUNK:
                pending = stage(k + 1, 1 - buf)

            def batch(jb, carry2, _buf=buf):
                j0 = jb * K
                gd = [pltpu.async_copy(
                        xp_hbm.at[srcv.at[_buf].at[pl.ds((j0 + b) * ROW, ROW)]],
                        rows.at[b], gsem)
                      for b in range(K)]
                sd = []
                for b in range(K):
                    gd[b].wait()
                    sd.append(pltpu.async_copy(
                        rows.at[b],
                        acc.at[dstv.at[_buf].at[pl.ds((j0 + b) * ROW, ROW)]],
                        ssem, add=True))
                for d in sd:
                    d.wait()
                return carry2

            lax.fori_loop(0, CH // K, batch, 0)
        plsc.subcore_barrier()
        pltpu.sync_copy(acc.at[pl.ds(s * RPS, RPS)],
                        out_hbm.at[c, pl.ds(s * RPS, RPS)])

    return sc_kernel(xp, zeros_acc, edge3)


FOLD = 8
NR = N_ACC * F // 128


def _tc_head(part_f, xp_f, w_big, b1_tile, watt_blk, esel, bbrd, w2_blk, b2):
    def tc_kernel(part_ref, xp_ref, wb_ref, b1_ref, wa_ref, es_ref, bb_ref,
                  w2_ref, b2_ref, out_ref):
        acc = part_ref[0] + part_ref[1] + xp_ref[...]
        den8 = jnp.dot(acc, es_ref[...],
                       preferred_element_type=jnp.float32)
        den = jnp.dot(den8, bb_ref[...],
                      preferred_element_type=jnp.float32)
        den = jnp.maximum(den, 0.5)
        num = jnp.dot(acc, wb_ref[...], preferred_element_type=jnp.float32)
        h = jnp.maximum(num / den + b1_ref[...], 0.0)
        t8 = jnp.tanh(jnp.dot(h, wa_ref[...],
                              preferred_element_type=jnp.float32))
        rid = lax.broadcasted_iota(jnp.int32, (NR, FOLD), 0)
        cid = lax.broadcasted_iota(jnp.int32, (NR, FOLD), 1)
        valid = rid * FOLD + cid < N
        t8 = jnp.where(valid, t8, -30.0)
        m = jnp.max(t8)
        e = jnp.exp(t8 - m)
        scores = e / jnp.sum(e)
        o8 = jnp.dot(h, w2_ref[...], preferred_element_type=jnp.float32)
        out_ref[...] = o8 * scores + b2_ref[...]

    return pl.pallas_call(
        tc_kernel,
        out_shape=jax.ShapeDtypeStruct((NR, FOLD), jnp.float32),
    )(part_f, xp_f, w_big, b1_tile, watt_blk, esel, bbrd, w2_blk, b2)


def kernel(x, edge_index, W1, b1, w_att, W2, b2):
    col = lax.broadcasted_iota(jnp.int32, (N_ACC, F), 1)
    xp = jnp.where(col == DEG_LANE, 1.0,
                   jnp.pad(x, ((0, N_ACC - N), (0, F - 11))))
    zeros_acc = jnp.zeros((N_ACC, F), jnp.float32)

    part = _sc_segment_sum(xp, zeros_acc, edge_index)

    w1p = jnp.zeros((F, F), jnp.float32).at[:11, :].set(W1)
    eye8 = jnp.eye(FOLD, dtype=jnp.float32)
    w_big = jnp.kron(eye8, w1p)
    watt_blk = jnp.kron(eye8, w_att.reshape(F, 1))
    esel = jnp.kron(eye8, jnp.zeros((F, 1), jnp.float32)
                    .at[DEG_LANE, 0].set(1.0))
    bbrd = jnp.kron(eye8, jnp.ones((1, F), jnp.float32))
    w2_blk = jnp.kron(eye8, W2)
    b1_tile = jnp.tile(b1, FOLD).reshape(1, 128)

    out = _tc_head(part.reshape(NC, NR, 128), xp.reshape(NR, 128),
                   w_big, b1_tile, watt_blk, esel, bbrd, w2_blk,
                   b2.reshape(1, 1))
    return out.reshape(N_ACC)[:N]


# speedup vs baseline: 62.5611x; 1.2015x over previous
import functools

import jax
import jax.numpy as jnp
from jax import lax
from jax.experimental import pallas as pl
from jax.experimental.pallas import tpu as pltpu
from jax.experimental.pallas import tpu_sc as plsc

N = 50000
E = 1600000
V = 11
F = 16
DEG_LANE = 11
NC = 2
NS = 16
NW = NC * NS
ROW = 400
CH = 25
CHE = CH * ROW
EPW = E // NW
NCHUNK = EPW // CHE
K = 6
DELTA = 3
N_ACC = ((N + NS - 1) // NS + 7) // 8 * 8 * NS
RPS = N_ACC // NS

NB = N // NW
NBX = N - NB * NW
L_STG = (NB + 1) * V + 7 + 32
L_STG = (L_STG + 7) // 8 * 8
X1_LEN = ((NW - 1) * NB + NBX) * V // 8 * 8 + L_STG


def _sc_build_table(x1):
    mesh = plsc.VectorSubcoreMesh(
        core_axis_name="c", subcore_axis_name="s", num_cores=NC)

    @functools.partial(
        pl.kernel,
        out_type=jax.ShapeDtypeStruct((N_ACC * F,), jnp.float32),
        mesh=mesh,
        scratch_types=[
            pltpu.VMEM((L_STG,), jnp.float32),
            pltpu.VMEM(((NB + 1) * F,), jnp.float32),
            pltpu.VMEM((64 * F,), jnp.float32),
        ],
        compiler_params=pltpu.CompilerParams(needs_layout_passes=False),
    )
    def build_kernel(x1_hbm, xp_hbm, stg, outb, zb):
        c = lax.axis_index("c")
        s = lax.axis_index("s")
        w = s * NC + c
        n0 = w * NB + jnp.minimum(w, NBX)
        w0 = n0 * V
        a0 = w0 // 8 * 8
        sh = w0 - a0
        pltpu.sync_copy(x1_hbm.at[pl.ds(a0, L_STG)], stg)
        lane = lax.broadcasted_iota(jnp.int32, (16,), 0)
        ones = jnp.full((16,), 1.0, jnp.float32)
        zero = jnp.zeros((16,), jnp.float32)

        def repack(j, carry):
            v = plsc.load_gather(stg, [sh + j * V + lane])
            rowv = jnp.where(lane < V, v, jnp.where(lane == DEG_LANE,
                                                    ones, zero))
            outb[pl.ds(j * F, F)] = rowv
            return carry

        lax.fori_loop(0, NB + 1, repack, 0)
        pltpu.sync_copy(outb.at[pl.ds(0, NB * F)],
                        xp_hbm.at[pl.ds(n0 * F, NB * F)])

        @pl.when(w < NBX)
        def _():
            pltpu.sync_copy(outb.at[pl.ds(NB * F, F)],
                            xp_hbm.at[pl.ds((n0 + NB) * F, F)])

        @pl.when(w == NW - 1)
        def _():
            def zrow(j, carry):
                zb[pl.ds(j * F, F)] = zero
                return carry
            lax.fori_loop(0, N_ACC - N, zrow, 0)
            pltpu.sync_copy(zb.at[pl.ds(0, (N_ACC - N) * F)],
                            xp_hbm.at[pl.ds(N * F, (N_ACC - N) * F)])

    return build_kernel(x1)


def _sc_segment_sum(xp2d, zeros_acc, edge2):
    mesh = plsc.VectorSubcoreMesh(
        core_axis_name="c", subcore_axis_name="s", num_cores=NC)

    @functools.partial(
        pl.kernel,
        out_type=jax.ShapeDtypeStruct((NC, N_ACC, F), jnp.float32),
        mesh=mesh,
        scratch_types=[
            pltpu.VMEM_SHARED((N_ACC, F), jnp.float32),
            pltpu.VMEM((2, CHE), jnp.int32),
            pltpu.VMEM((2, CHE), jnp.int32),
            pltpu.VMEM((K, ROW, F), jnp.float32),
            pltpu.SemaphoreType.DMA,
            pltpu.SemaphoreType.DMA,
            pltpu.SemaphoreType.DMA,
        ],
        compiler_params=pltpu.CompilerParams(use_tc_tiling_on_sc=False),
    )
    def sc_kernel(xp_hbm, zer_hbm, edge_hbm, out_hbm,
                  acc, srcv, dstv, rows, gsem, ssem, esem):
        c = lax.axis_index("c")
        s = lax.axis_index("s")
        w = s * NC + c

        def stage(k, buf):
            base = w * EPW + k * CHE
            return [pltpu.async_copy(edge_hbm.at[0, pl.ds(base, CHE)],
                                     srcv.at[buf], esem),
                    pltpu.async_copy(edge_hbm.at[1, pl.ds(base, CHE)],
                                     dstv.at[buf], esem)]

        pending = stage(0, 0)
        @pl.when(c == 0)
        def _():
            pltpu.sync_copy(xp_hbm.at[pl.ds(s * RPS, RPS)],
                            acc.at[pl.ds(s * RPS, RPS)])

        @pl.when(c == 1)
        def _():
            pltpu.sync_copy(zer_hbm.at[pl.ds(s * RPS, RPS)],
                            acc.at[pl.ds(s * RPS, RPS)])
        plsc.subcore_barrier()

        def drain(sem):
            pltpu.make_async_copy(xp_hbm.at[pl.ds(0, ROW)],
                                  rows.at[0], sem).wait()

        for k in range(NCHUNK):
            buf = k % 2
            for d in pending:
                d.wait()
            if k + 1 < NCHUNK:
                pending = stage(k + 1, 1 - buf)

            def step(j, carry, _buf=buf):
                @pl.when(j >= K)
                def _():
                    drain(ssem)
                pltpu.async_copy(
                    xp_hbm.at[srcv.at[_buf].at[pl.ds(j * ROW, ROW)]],
                    rows.at[lax.rem(j, K)], gsem)

                @pl.when(j >= DELTA)
                def _():
                    jd = j - DELTA
                    drain(gsem)
                    pltpu.async_copy(
                        rows.at[lax.rem(jd, K)],
                        acc.at[dstv.at[_buf].at[pl.ds(jd * ROW, ROW)]],
                        ssem, add=True)
                return carry

            lax.fori_loop(0, CH, step, 0)

            def epilogue(j, carry, _buf=buf):
                drain(gsem)
                pltpu.async_copy(
                    rows.at[lax.rem(j, K)],
                    acc.at[dstv.at[_buf].at[pl.ds(j * ROW, ROW)]],
                    ssem, add=True)
                return carry

            lax.fori_loop(CH - DELTA, CH, epilogue, 0)

            def flush(j, carry):
                drain(ssem)
                return carry

            lax.fori_loop(0, K, flush, 0)
        plsc.subcore_barrier()
        pltpu.sync_copy(acc.at[pl.ds(s * RPS, RPS)],
                        out_hbm.at[c, pl.ds(s * RPS, RPS)])

    return sc_kernel(xp2d, zeros_acc, edge2)


FOLD = 8
NR = N_ACC * F // 128


def _tc_head(part_f, w_big, b1_tile, watt_blk, esel, bbrd, w2_blk, b2):
    def tc_kernel(part_ref, wb_ref, b1_ref, wa_ref, es_ref, bb_ref,
                  w2_ref, b2_ref, out_ref):
        acc = part_ref[0] + part_ref[1]
        den8 = jnp.dot(acc, es_ref[...],
                       preferred_element_type=jnp.float32)
        den = jnp.dot(den8, bb_ref[...],
                      preferred_element_type=jnp.float32)
        den = jnp.maximum(den, 0.5)
        num = jnp.dot(acc, wb_ref[...], preferred_element_type=jnp.float32)
        h = jnp.maximum(num / den + b1_ref[...], 0.0)
        t8 = jnp.tanh(jnp.dot(h, wa_ref[...],
                              preferred_element_type=jnp.float32))
        rid = lax.broadcasted_iota(jnp.int32, (NR, FOLD), 0)
        cid = lax.broadcasted_iota(jnp.int32, (NR, FOLD), 1)
        valid = rid * FOLD + cid < N
        t8 = jnp.where(valid, t8, -30.0)
        m = jnp.max(t8)
        e = jnp.exp(t8 - m)
        scores = e / jnp.sum(e)
        o8 = jnp.dot(h, w2_ref[...], preferred_element_type=jnp.float32)
        out_ref[...] = o8 * scores + b2_ref[...]

    return pl.pallas_call(
        tc_kernel,
        out_shape=jax.ShapeDtypeStruct((NR, FOLD), jnp.float32),
    )(part_f, w_big, b1_tile, watt_blk, esel, bbrd, w2_blk, b2)


def kernel(x, edge_index, W1, b1, w_att, W2, b2):
    x1 = jnp.pad(x.reshape(-1), (0, X1_LEN - N * V))
    xpflat = _sc_build_table(x1)
    xp2d = xpflat.reshape(N_ACC, F)
    zeros_acc = jnp.zeros((N_ACC, F), jnp.float32)

    part = _sc_segment_sum(xp2d, zeros_acc, edge_index)

    w1p = jnp.zeros((F, F), jnp.float32).at[:11, :].set(W1)
    eye8 = jnp.eye(FOLD, dtype=jnp.float32)
    w_big = jnp.kron(eye8, w1p)
    watt_blk = jnp.kron(eye8, w_att.reshape(F, 1))
    esel = jnp.kron(eye8, jnp.zeros((F, 1), jnp.float32)
                    .at[DEG_LANE, 0].set(1.0))
    bbrd = jnp.kron(eye8, jnp.ones((1, F), jnp.float32))
    w2_blk = jnp.kron(eye8, W2)
    b1_tile = jnp.tile(b1, FOLD).reshape(1, 128)

    out = _tc_head(part.reshape(NC, NR, 128),
                   w_big, b1_tile, watt_blk, esel, bbrd, w2_blk,
                   b2.reshape(1, 1))
    return out.reshape(N_ACC)[:N]
